# baseline (speedup 1.0000x reference)
import functools

import jax
import jax.numpy as jnp
from jax import lax
from jax.experimental import pallas as pl
from jax.experimental.pallas import tpu as pltpu
from jax.experimental.pallas import tpu_sc as plsc

_B = 4096
_D = 32
_NCAT = 26
_NNUM = 13
_CATV = 100000
_LA = 200
_LS = 50

_NC = 2
_NS = 16
_NW = _NC * _NS
_BPW = _B // _NW

_ATOM_K, _ATOM_NT = 20, 10
_SEM_K, _SEM_NT = 10, 5


def _sc_cat_body(cat_tab3, catT, cat_out3, rows3, idxc_v, dsem):
    w = lax.axis_index("s") * _NC + lax.axis_index("c")
    b0 = w * _BPW
    pltpu.sync_copy(catT.at[:, pl.ds(b0, _BPW)], idxc_v)
    cdescs = [
        pltpu.async_copy(cat_tab3.at[f].at[idxc_v.at[f]], rows3.at[f], dsem)
        for f in range(_NCAT)
    ]
    for d in cdescs:
        d.wait()
    pltpu.sync_copy(rows3, cat_out3.at[:, pl.ds(b0, _BPW), :])


def _sc_cat(cat_tab3, catT):
    kfn = pl.kernel(
        _sc_cat_body,
        out_type=jax.ShapeDtypeStruct((_NCAT, _B, _D), jnp.float32),
        mesh=plsc.VectorSubcoreMesh(core_axis_name="c", subcore_axis_name="s",
                                    num_cores=_NC, num_subcores=_NS),
        scratch_types=[
            pltpu.VMEM((_NCAT, 128, _D), jnp.float32),
            pltpu.VMEM((_NCAT, 128), jnp.int32),
            pltpu.SemaphoreType.DMA,
        ],
        compiler_params=pltpu.CompilerParams(use_tc_tiling_on_sc=False),
    )
    return kfn(cat_tab3, catT)


def _sc_hist_body(atom_tab, atom_idx, sem_tab, sem_idx,
                  atom_out3, sem_out3, rows3, idx_v, dsem):
    w = lax.axis_index("s") * _NC + lax.axis_index("c")
    for tab, idx1, out3, K, nt in (
            (atom_tab, atom_idx, atom_out3, _ATOM_K, _ATOM_NT),
            (sem_tab, sem_idx, sem_out3, _SEM_K, _SEM_NT)):

        def tile(t, carry, tab=tab, idx1=idx1, out3=out3, K=K, nt=nt):
            r0 = (w * nt + t) * K
            pltpu.sync_copy(idx1.at[pl.ds(r0 * 128, K * 128)],
                            idx_v.at[pl.ds(0, K * 128)])
            descs = [
                pltpu.async_copy(tab.at[idx_v.at[pl.ds(j * 128, 128)]],
                                 rows3.at[j], dsem)
                for j in range(K)
            ]
            for d in descs:
                d.wait()
            pltpu.sync_copy(rows3.at[pl.ds(0, K)], out3.at[pl.ds(r0, K)])
            return carry

        lax.fori_loop(0, nt, tile, 0)


def _sc_hist(atom_tab, atom_idx, sem_tab, sem_idx):
    kfn = pl.kernel(
        _sc_hist_body,
        out_type=(
            jax.ShapeDtypeStruct((_B * _LA // 128, 128, _D), jnp.float32),
            jax.ShapeDtypeStruct((_B * _LS // 128, 128, _D), jnp.float32),
        ),
        mesh=plsc.VectorSubcoreMesh(core_axis_name="c", subcore_axis_name="s",
                                    num_cores=_NC, num_subcores=_NS),
        scratch_types=[
            pltpu.VMEM((_ATOM_K, 128, _D), jnp.float32),
            pltpu.VMEM((_ATOM_K * 128,), jnp.int32),
            pltpu.SemaphoreType.DMA,
        ],
        compiler_params=pltpu.CompilerParams(use_tc_tiling_on_sc=False),
    )
    return kfn(atom_tab, atom_idx, sem_tab, sem_idx)


def _gelu_exact(x):
    return 0.5 * x * (1.0 + lax.erf(x * 0.7071067811865476))


def _layernorm_last(x, g, b, eps=1e-5):
    mu = jnp.mean(x, axis=-1, keepdims=True)
    var = jnp.mean((x - mu) ** 2, axis=-1, keepdims=True)
    return (x - mu) * lax.rsqrt(var + eps) * g + b


def _mlp_body(cat_ref, nf_ref, nw_ref, nb_ref, W_ref, b_ref, g_ref, be_ref,
              fc_ref, fn_ref):
    W = W_ref[...]
    b = b_ref[...]
    g = g_ref[...]
    be = be_ref[...]
    h = jnp.dot(cat_ref[...], W, preferred_element_type=jnp.float32) + b
    fc_ref[...] = _layernorm_last(_gelu_exact(h), g, be)
    A = jnp.dot(nw_ref[...], W, preferred_element_type=jnp.float32)
    C = jnp.dot(nb_ref[...], W, preferred_element_type=jnp.float32) + b
    f = nf_ref[...]
    hn = f[:, :, None] * A[None, :, :] + C[None, :, :]
    fn_ref[...] = _layernorm_last(_gelu_exact(hn), g[None], be[None])


def _mlp_tc(cat_emb2, num_feats, num_w, num_b, mlp_W, mlp_b2, ln_g2, ln_b2):
    BB = 512
    CB = BB * _NCAT
    grid = (_B // BB,)
    return pl.pallas_call(
        _mlp_body,
        grid=grid,
        in_specs=[
            pl.BlockSpec((CB, _D), lambda i: (i, 0)),
            pl.BlockSpec((BB, _NNUM), lambda i: (i, 0)),
            pl.BlockSpec((_NNUM, _D), lambda i: (0, 0)),
            pl.BlockSpec((_NNUM, _D), lambda i: (0, 0)),
            pl.BlockSpec((_D, _D), lambda i: (0, 0)),
            pl.BlockSpec((1, _D), lambda i: (0, 0)),
            pl.BlockSpec((1, _D), lambda i: (0, 0)),
            pl.BlockSpec((1, _D), lambda i: (0, 0)),
        ],
        out_specs=[
            pl.BlockSpec((CB, _D), lambda i: (i, 0)),
            pl.BlockSpec((BB, _NNUM, _D), lambda i: (i, 0, 0)),
        ],
        out_shape=[
            jax.ShapeDtypeStruct((_B * _NCAT, _D), jnp.float32),
            jax.ShapeDtypeStruct((_B, _NNUM, _D), jnp.float32),
        ],
    )(cat_emb2, num_feats, num_w, num_b, mlp_W, mlp_b2, ln_g2, ln_b2)


def kernel(cat_feats, num_feats, atom_history, sem_history, cat_tables, num_w,
           num_b, mlp_W, mlp_b, ln_gamma, ln_beta, atom_table, sem_table):
    catT = cat_feats.T
    atom_idx = atom_history.reshape(_B * _LA)
    sem_idx = sem_history.reshape(_B * _LS)

    atom_tok3, sem_tok3 = _sc_hist(atom_table, atom_idx, sem_table, sem_idx)
    cat_fm3 = _sc_cat(cat_tables, catT)

    feat_cat2, feat_num = _mlp_tc(
        cat_fm3.reshape(_NCAT * _B, _D), num_feats, num_w, num_b, mlp_W,
        mlp_b.reshape(1, _D), ln_gamma.reshape(1, _D), ln_beta.reshape(1, _D))

    feat_cat = feat_cat2.reshape(_NCAT, _B, _D).transpose(1, 0, 2)
    return jnp.concatenate([
        feat_cat,
        feat_num,
        atom_tok3.reshape(_B, _LA, _D),
        sem_tok3.reshape(_B, _LS, _D),
    ], axis=1)


# speedup vs baseline: 1.0006x; 1.0006x over previous
import functools

import jax
import jax.numpy as jnp
from jax import lax
from jax.experimental import pallas as pl
from jax.experimental.pallas import tpu as pltpu
from jax.experimental.pallas import tpu_sc as plsc

_B = 4096
_D = 32
_NCAT = 26
_NNUM = 13
_CATV = 100000
_LA = 200
_LS = 50

_NC = 2
_NS = 16
_NW = _NC * _NS
_BPW = _B // _NW

_ATOM_K, _ATOM_NT = 20, 10
_SEM_K, _SEM_NT = 10, 5


def _sc_cat_body(cat_tab3, catT, cat_out3, rows3, idxc_v, dsem):
    w = lax.axis_index("s") * _NC + lax.axis_index("c")
    b0 = w * _BPW
    pltpu.sync_copy(catT.at[:, pl.ds(b0, _BPW)], idxc_v)
    cdescs = [
        pltpu.async_copy(cat_tab3.at[f].at[idxc_v.at[f]], rows3.at[f], dsem)
        for f in range(_NCAT)
    ]
    for d in cdescs:
        d.wait()
    pltpu.sync_copy(rows3, cat_out3.at[:, pl.ds(b0, _BPW), :])


def _sc_cat(cat_tab3, catT):
    kfn = pl.kernel(
        _sc_cat_body,
        out_type=jax.ShapeDtypeStruct((_NCAT, _B, _D), jnp.float32),
        mesh=plsc.VectorSubcoreMesh(core_axis_name="c", subcore_axis_name="s",
                                    num_cores=_NC, num_subcores=_NS),
        scratch_types=[
            pltpu.VMEM((_NCAT, 128, _D), jnp.float32),
            pltpu.VMEM((_NCAT, 128), jnp.int32),
            pltpu.SemaphoreType.DMA,
        ],
        compiler_params=pltpu.CompilerParams(use_tc_tiling_on_sc=False),
    )
    return kfn(cat_tab3, catT)


def _sc_hist_body(atom_tab, atom_idx, sem_tab, sem_idx,
                  atom_out3, sem_out3, rows3, idx_v, dsem):
    w = lax.axis_index("s") * _NC + lax.axis_index("c")
    for tab, idx1, out3, K, nt in (
            (atom_tab, atom_idx, atom_out3, _ATOM_K, _ATOM_NT),
            (sem_tab, sem_idx, sem_out3, _SEM_K, _SEM_NT)):

        def tile(t, carry, tab=tab, idx1=idx1, out3=out3, K=K, nt=nt):
            r0 = (w * nt + t) * K
            pltpu.sync_copy(idx1.at[pl.ds(r0 * 128, K * 128)],
                            idx_v.at[pl.ds(0, K * 128)])
            descs = [
                pltpu.async_copy(tab.at[idx_v.at[pl.ds(j * 128, 128)]],
                                 rows3.at[j], dsem)
                for j in range(K)
            ]
            for d in descs:
                d.wait()
            pltpu.sync_copy(rows3.at[pl.ds(0, K)], out3.at[pl.ds(r0, K)])
            return carry

        lax.fori_loop(0, nt, tile, 0)


def _sc_hist(atom_tab, atom_idx, sem_tab, sem_idx):
    kfn = pl.kernel(
        _sc_hist_body,
        out_type=(
            jax.ShapeDtypeStruct((_B * _LA // 128, 128, _D), jnp.float32),
            jax.ShapeDtypeStruct((_B * _LS // 128, 128, _D), jnp.float32),
        ),
        mesh=plsc.VectorSubcoreMesh(core_axis_name="c", subcore_axis_name="s",
                                    num_cores=_NC, num_subcores=_NS),
        scratch_types=[
            pltpu.VMEM((_ATOM_K, 128, _D), jnp.float32),
            pltpu.VMEM((_ATOM_K * 128,), jnp.int32),
            pltpu.SemaphoreType.DMA,
        ],
        compiler_params=pltpu.CompilerParams(use_tc_tiling_on_sc=False),
    )
    return kfn(atom_tab, atom_idx, sem_tab, sem_idx)


def _gelu_exact(x):
    return 0.5 * x * (1.0 + lax.erf(x * 0.7071067811865476))


def _layernorm_last(x, g, b, eps=1e-5):
    mu = jnp.mean(x, axis=-1, keepdims=True)
    var = jnp.mean((x - mu) ** 2, axis=-1, keepdims=True)
    return (x - mu) * lax.rsqrt(var + eps) * g + b


def _mlp_body(cat_ref, nf_ref, nw_ref, nb_ref, W_ref, b_ref, g_ref, be_ref,
              fc_ref, fn_ref):
    W = W_ref[...]
    b = b_ref[...]
    g = g_ref[...]
    be = be_ref[...]
    h = jnp.dot(cat_ref[...], W, preferred_element_type=jnp.float32) + b
    fc_ref[...] = _layernorm_last(_gelu_exact(h), g, be)
    A = jnp.dot(nw_ref[...], W, preferred_element_type=jnp.float32)
    C = jnp.dot(nb_ref[...], W, preferred_element_type=jnp.float32) + b
    f = nf_ref[...]
    hn = f[:, :, None] * A[None, :, :] + C[None, :, :]
    fn_ref[...] = _layernorm_last(_gelu_exact(hn), g[None], be[None])


def _mlp_tc(cat_emb2, num_feats, num_w, num_b, mlp_W, mlp_b2, ln_g2, ln_b2):
    BB = 512
    CB = BB * _NCAT
    grid = (_B // BB,)
    return pl.pallas_call(
        _mlp_body,
        grid=grid,
        in_specs=[
            pl.BlockSpec((CB, _D), lambda i: (i, 0)),
            pl.BlockSpec((BB, _NNUM), lambda i: (i, 0)),
            pl.BlockSpec((_NNUM, _D), lambda i: (0, 0)),
            pl.BlockSpec((_NNUM, _D), lambda i: (0, 0)),
            pl.BlockSpec((_D, _D), lambda i: (0, 0)),
            pl.BlockSpec((1, _D), lambda i: (0, 0)),
            pl.BlockSpec((1, _D), lambda i: (0, 0)),
            pl.BlockSpec((1, _D), lambda i: (0, 0)),
        ],
        out_specs=[
            pl.BlockSpec((CB, _D), lambda i: (i, 0)),
            pl.BlockSpec((BB, _NNUM, _D), lambda i: (i, 0, 0)),
        ],
        out_shape=[
            jax.ShapeDtypeStruct((_B * _NCAT, _D), jnp.float32),
            jax.ShapeDtypeStruct((_B, _NNUM, _D), jnp.float32),
        ],
    )(cat_emb2, num_feats, num_w, num_b, mlp_W, mlp_b2, ln_g2, ln_b2)


def kernel(cat_feats, num_feats, atom_history, sem_history, cat_tables, num_w,
           num_b, mlp_W, mlp_b, ln_gamma, ln_beta, atom_table, sem_table):
    catT = cat_feats.T
    atom_idx = atom_history.reshape(_B * _LA)
    sem_idx = sem_history.reshape(_B * _LS)

    cat_fm3 = _sc_cat(cat_tables, catT)
    atom_tok3, sem_tok3 = _sc_hist(atom_table, atom_idx, sem_table, sem_idx)

    feat_cat2, feat_num = _mlp_tc(
        cat_fm3.reshape(_NCAT * _B, _D), num_feats, num_w, num_b, mlp_W,
        mlp_b.reshape(1, _D), ln_gamma.reshape(1, _D), ln_beta.reshape(1, _D))

    feat_cat = feat_cat2.reshape(_NCAT, _B, _D).transpose(1, 0, 2)
    return jnp.concatenate([
        feat_cat,
        feat_num,
        atom_tok3.reshape(_B, _LA, _D),
        sem_tok3.reshape(_B, _LS, _D),
    ], axis=1)


# speedup vs baseline: 1.0977x; 1.0970x over previous
import functools

import jax
import jax.numpy as jnp
from jax import lax
from jax.experimental import pallas as pl
from jax.experimental.pallas import tpu as pltpu
from jax.experimental.pallas import tpu_sc as plsc

_B = 4096
_D = 32
_NCAT = 26
_NNUM = 13
_CATV = 100000
_LA = 200
_LS = 50

_NC = 2
_NS = 16
_NW = _NC * _NS
_BPW = _B // _NW

_ATOM_K, _ATOM_NT = 20, 10
_SEM_K, _SEM_NT = 10, 5


def _sc_gather_body(cat_tab3, catT, atom_tab, atomT, sem_tab, semT,
                    cat_out3, atom_out3, sem_out3, rows3, idx_v, idxc_v, dsem):
    w = lax.axis_index("s") * _NC + lax.axis_index("c")
    b0 = w * _BPW

    pltpu.sync_copy(catT.at[:, pl.ds(b0, _BPW)], idxc_v)
    cdescs = [
        pltpu.async_copy(cat_tab3.at[f].at[idxc_v.at[f]], rows3.at[f], dsem)
        for f in range(_NCAT)
    ]
    for d in cdescs:
        d.wait()
    pltpu.sync_copy(rows3, cat_out3.at[:, pl.ds(b0, _BPW), :])

    for half in range(2):
        t0 = half * 100
        pltpu.sync_copy(atomT.at[pl.ds(t0, 100), pl.ds(b0, _BPW)], idx_v)

        def agrp(g, carry, t0=t0):
            descs = [
                pltpu.async_copy(atom_tab.at[idx_v.at[g * _ATOM_K + j]],
                                 rows3.at[j], dsem)
                for j in range(_ATOM_K)
            ]
            for d in descs:
                d.wait()
            odescs = [
                pltpu.async_copy(
                    rows3.at[j],
                    atom_out3.at[t0 + g * _ATOM_K + j, pl.ds(b0, _BPW)], dsem)
                for j in range(_ATOM_K)
            ]
            for d in odescs:
                d.wait()
            return carry

        lax.fori_loop(0, 100 // _ATOM_K, agrp, 0)

    pltpu.sync_copy(semT.at[:, pl.ds(b0, _BPW)], idx_v.at[pl.ds(0, _LS)])

    def sgrp(g, carry):
        descs = [
            pltpu.async_copy(sem_tab.at[idx_v.at[g * _SEM_K + j]],
                             rows3.at[j], dsem)
            for j in range(_SEM_K)
        ]
        for d in descs:
            d.wait()
        odescs = [
            pltpu.async_copy(rows3.at[j],
                             sem_out3.at[g * _SEM_K + j, pl.ds(b0, _BPW)],
                             dsem)
            for j in range(_SEM_K)
        ]
        for d in odescs:
            d.wait()
        return carry

    lax.fori_loop(0, _LS // _SEM_K, sgrp, 0)


def _sc_gather_all(cat_tab3, catT, atom_tab, atomT, sem_tab, semT):
    kfn = pl.kernel(
        _sc_gather_body,
        out_type=(
            jax.ShapeDtypeStruct((_NCAT, _B, _D), jnp.float32),
            jax.ShapeDtypeStruct((_LA, _B, _D), jnp.float32),
            jax.ShapeDtypeStruct((_LS, _B, _D), jnp.float32),
        ),
        mesh=plsc.VectorSubcoreMesh(core_axis_name="c", subcore_axis_name="s",
                                    num_cores=_NC, num_subcores=_NS),
        scratch_types=[
            pltpu.VMEM((_NCAT, 128, _D), jnp.float32),
            pltpu.VMEM((100, 128), jnp.int32),
            pltpu.VMEM((_NCAT, 128), jnp.int32),
            pltpu.SemaphoreType.DMA,
        ],
        compiler_params=pltpu.CompilerParams(use_tc_tiling_on_sc=False),
    )
    return kfn(cat_tab3, catT, atom_tab, atomT, sem_tab, semT)


def _gelu_exact(x):
    return 0.5 * x * (1.0 + lax.erf(x * 0.7071067811865476))


def _layernorm_last(x, g, b, eps=1e-5):
    mu = jnp.mean(x, axis=-1, keepdims=True)
    var = jnp.mean((x - mu) ** 2, axis=-1, keepdims=True)
    return (x - mu) * lax.rsqrt(var + eps) * g + b


def _mlp_body(cat_ref, nf_ref, nw_ref, nb_ref, W_ref, b_ref, g_ref, be_ref,
              fc_ref, fn_ref):
    W = W_ref[...]
    b = b_ref[...]
    g = g_ref[...]
    be = be_ref[...]
    h = jnp.dot(cat_ref[...], W, preferred_element_type=jnp.float32) + b
    fc_ref[...] = _layernorm_last(_gelu_exact(h), g, be)
    A = jnp.dot(nw_ref[...], W, preferred_element_type=jnp.float32)
    C = jnp.dot(nb_ref[...], W, preferred_element_type=jnp.float32) + b
    f = nf_ref[...]
    hn = f[:, :, None] * A[None, :, :] + C[None, :, :]
    fn_ref[...] = _layernorm_last(_gelu_exact(hn), g[None], be[None])


def _mlp_tc(cat_emb2, num_feats, num_w, num_b, mlp_W, mlp_b2, ln_g2, ln_b2):
    BB = 512
    CB = BB * _NCAT
    grid = (_B // BB,)
    return pl.pallas_call(
        _mlp_body,
        grid=grid,
        in_specs=[
            pl.BlockSpec((CB, _D), lambda i: (i, 0)),
            pl.BlockSpec((BB, _NNUM), lambda i: (i, 0)),
            pl.BlockSpec((_NNUM, _D), lambda i: (0, 0)),
            pl.BlockSpec((_NNUM, _D), lambda i: (0, 0)),
            pl.BlockSpec((_D, _D), lambda i: (0, 0)),
            pl.BlockSpec((1, _D), lambda i: (0, 0)),
            pl.BlockSpec((1, _D), lambda i: (0, 0)),
            pl.BlockSpec((1, _D), lambda i: (0, 0)),
        ],
        out_specs=[
            pl.BlockSpec((CB, _D), lambda i: (i, 0)),
            pl.BlockSpec((BB, _NNUM, _D), lambda i: (i, 0, 0)),
        ],
        out_shape=[
            jax.ShapeDtypeStruct((_B * _NCAT, _D), jnp.float32),
            jax.ShapeDtypeStruct((_B, _NNUM, _D), jnp.float32),
        ],
    )(cat_emb2, num_feats, num_w, num_b, mlp_W, mlp_b2, ln_g2, ln_b2)


def kernel(cat_feats, num_feats, atom_history, sem_history, cat_tables, num_w,
           num_b, mlp_W, mlp_b, ln_gamma, ln_beta, atom_table, sem_table):
    catT = cat_feats.T
    atomT = atom_history.T
    semT = sem_history.T

    cat_fm3, atom_tok3, sem_tok3 = _sc_gather_all(
        cat_tables, catT, atom_table, atomT, sem_table, semT)

    feat_cat2, feat_num = _mlp_tc(
        cat_fm3.reshape(_NCAT * _B, _D), num_feats, num_w, num_b, mlp_W,
        mlp_b.reshape(1, _D), ln_gamma.reshape(1, _D), ln_beta.reshape(1, _D))

    feat_cat = feat_cat2.reshape(_NCAT, _B, _D).transpose(1, 0, 2)
    return jnp.concatenate([
        feat_cat,
        feat_num,
        atom_tok3.transpose(1, 0, 2),
        sem_tok3.transpose(1, 0, 2),
    ], axis=1)
